# baseline (speedup 1.0000x reference)
import functools

import jax
import jax.numpy as jnp
from jax import lax
from jax.experimental import pallas as pl
from jax.experimental.pallas import tpu as pltpu
from jax.experimental.pallas import tpu_sc as plsc

TEMP = 0.07
Q = 1024
DIM = 16
K = 100000
NW = 32
SLAB = 3136
SLABP = SLAB + 16
TILE = 32


def _sc_body(xt2_hbm, xb_hbm, yt_hbm, ysb_hbm, out_hbm,
             xt2_v, xb_v, yt_v, ysb_v, row_v):
    c = lax.axis_index("c")
    s = lax.axis_index("s")
    wid = s * 2 + c
    k0 = jnp.minimum(wid * SLAB, K - SLAB)

    pltpu.sync_copy(xt2_hbm, xt2_v)
    pltpu.sync_copy(xb_hbm, xb_v)
    for d in range(DIM):
        pltpu.sync_copy(yt_hbm.at[pl.ds(d * K + k0, SLAB)],
                        yt_v.at[pl.ds(d * SLABP, SLAB)])
    pltpu.sync_copy(ysb_hbm.at[pl.ds(k0, SLAB)],
                    ysb_v.at[pl.ds(0, SLAB)])

    def tile_body(j, carry):
        def row_body(r, carry2):
            k = j * TILE + r
            bias = jnp.full((16,), ysb_v[pl.ds(k, 16)][0], jnp.float32)
            ybs = [jnp.full((16,), yt_v[pl.ds(d * SLABP + k, 16)][0],
                            jnp.float32)
                   for d in range(DIM)]
            for c2 in range(Q // 16):
                acc = xb_v[pl.ds(c2 * 16, 16)] + bias
                for d in range(DIM):
                    acc = acc + xt2_v[pl.ds(d * Q + c2 * 16, 16)] * ybs[d]
                row_v[r, pl.ds(c2 * 16, 16)] = jnp.minimum(acc, 0.0)
            return carry2
        lax.fori_loop(0, TILE, row_body, 0, unroll=False)
        pltpu.sync_copy(row_v, out_hbm.at[pl.ds(k0 + j * TILE, TILE), :])
        return carry
    lax.fori_loop(0, SLAB // TILE, tile_body, 0, unroll=False)


@jax.jit
def kernel(inputs, context):
    q, dim = inputs.shape
    k = context.shape[0]
    x_sq = jnp.sum(inputs * inputs, axis=1)
    y_sq = jnp.sum(context * context, axis=1)
    xt2 = (inputs.T * (2.0 / TEMP)).reshape(-1)
    xb = -x_sq / TEMP
    yt = context.T.reshape(-1)
    ysb = -y_sq / TEMP
    mesh = plsc.VectorSubcoreMesh(core_axis_name="c", subcore_axis_name="s")
    out_t = pl.kernel(
        _sc_body,
        mesh=mesh,
        out_type=jax.ShapeDtypeStruct((k, q), jnp.float32),
        compiler_params=pltpu.CompilerParams(use_tc_tiling_on_sc=True),
        scratch_types=[
            pltpu.VMEM((dim * q,), jnp.float32),
            pltpu.VMEM((q,), jnp.float32),
            pltpu.VMEM((dim * SLABP,), jnp.float32),
            pltpu.VMEM((SLABP,), jnp.float32),
            pltpu.VMEM((TILE, q), jnp.float32),
        ],
    )(xt2, xb, yt, ysb)
    return out_t.T
